# baseline (speedup 1.0000x reference)
import jax
import jax.numpy as jnp
from jax.experimental import pallas as pl

_D_IN = 2048
_BASE = 256
_NUM_LATENTS = 65536
_TB = 32


def _body(x_ref, w_ref, b_ref, out_ref):
    h = jax.lax.dot_general(
        x_ref[...], w_ref[...],
        dimension_numbers=(((1,), (1,)), ((), ())),
        preferred_element_type=jnp.float32,
    ) + b_ref[...]
    x1 = h[:, :_BASE]
    x2 = h[:, _BASE:]
    for k in range(_BASE):
        out_ref[:, k * _BASE:(k + 1) * _BASE] = x1[:, k:k + 1] + x2


def kernel(x, W, b):
    n_tok = x.shape[0]
    out = pl.pallas_call(
        _body,
        grid=(n_tok // _TB,),
        in_specs=[
            pl.BlockSpec((_TB, _D_IN), lambda t: (t, 0)),
            pl.BlockSpec((2 * _BASE, _D_IN), lambda t: (0, 0)),
            pl.BlockSpec((1, 2 * _BASE), lambda t: (0, 0)),
        ],
        out_specs=pl.BlockSpec((_TB, _BASE * _BASE), lambda t: (t, 0)),
        out_shape=jax.ShapeDtypeStruct((n_tok, _BASE * _BASE), jnp.float32),
    )(x, W, b.reshape(1, 2 * _BASE))
    return out[:, :_NUM_LATENTS]


# speedup vs baseline: 1.0980x; 1.0980x over previous
import jax
import jax.numpy as jnp
from jax.experimental import pallas as pl

_D_IN = 2048
_BASE = 256
_NUM_LATENTS = 65536
_TB = 64


def _body(x_ref, w_ref, b_ref, out_ref):
    h = jax.lax.dot_general(
        x_ref[...], w_ref[...],
        dimension_numbers=(((1,), (1,)), ((), ())),
        preferred_element_type=jnp.float32,
    ) + b_ref[...]
    x1 = h[:, :_BASE]
    x2 = h[:, _BASE:]
    for k in range(_BASE):
        out_ref[:, k * _BASE:(k + 1) * _BASE] = x1[:, k:k + 1] + x2


def kernel(x, W, b):
    n_tok = x.shape[0]
    out = pl.pallas_call(
        _body,
        grid=(n_tok // _TB,),
        in_specs=[
            pl.BlockSpec((_TB, _D_IN), lambda t: (t, 0)),
            pl.BlockSpec((2 * _BASE, _D_IN), lambda t: (0, 0)),
            pl.BlockSpec((1, 2 * _BASE), lambda t: (0, 0)),
        ],
        out_specs=pl.BlockSpec((_TB, _BASE * _BASE), lambda t: (t, 0)),
        out_shape=jax.ShapeDtypeStruct((n_tok, _BASE * _BASE), jnp.float32),
    )(x, W, b.reshape(1, 2 * _BASE))
    return out[:, :_NUM_LATENTS]
